# baseline (speedup 1.0000x reference)
import functools

import jax
import jax.numpy as jnp
from jax import lax
from jax.experimental import pallas as pl
from jax.experimental.pallas import tpu as pltpu
from jax.experimental.pallas import tpu_sc as plsc

_N = 10000
_E = 320000
_D = 128

_NC = 2
_NS = 16
_NW = _NC * _NS
_CHUNK = 80
_CPT = 128
_EPAD = _NW * _CPT * _CHUNK
_NPAD = 10240
_RPT = _NPAD // _NS
_NBUF = 4
_STAGE = 32

_mesh = plsc.VectorSubcoreMesh(core_axis_name="c", subcore_axis_name="s")


@functools.partial(
    pl.kernel,
    out_type=jax.ShapeDtypeStruct((_NC, _NPAD, _D), jnp.float32),
    mesh=_mesh,
    scratch_types=[
        pltpu.VMEM((_STAGE, _CHUNK), jnp.int32),
        pltpu.VMEM((_STAGE, _CHUNK), jnp.int32),
        [pltpu.VMEM((_CHUNK, _D), jnp.float32)] * _NBUF,
        pltpu.VMEM_SHARED((_NPAD, _D), jnp.float32),
        [pltpu.SemaphoreType.DMA] * _NBUF,
    ],
)
def _segsum_sc(h_hbm, src_hbm, dst_hbm, zeros_hbm, out_hbm,
               sidx, didx, rows, acc, gsem):
    cid = lax.axis_index("c")
    sid = lax.axis_index("s")
    w = cid * _NS + sid

    def _stage(st):
        pltpu.sync_copy(src_hbm.at[w, pl.ds(st * _STAGE, _STAGE)], sidx)
        pltpu.sync_copy(dst_hbm.at[w, pl.ds(st * _STAGE, _STAGE)], didx)
        for b in range(_NBUF):
            pltpu.async_copy(h_hbm.at[sidx.at[b]], rows[b], gsem[b])

    def _drain(st):
        @pl.loop(0, _STAGE - _NBUF, step=_NBUF)
        def _chunks(j):
            for b in range(_NBUF):
                pltpu.make_async_copy(h_hbm.at[sidx.at[j + b]],
                                      rows[b], gsem[b]).wait()
                pltpu.sync_copy(rows[b], acc.at[didx.at[j + b]], add=True)
                pltpu.async_copy(h_hbm.at[sidx.at[j + _NBUF + b]],
                                 rows[b], gsem[b])
        for b in range(_NBUF):
            j = _STAGE - _NBUF + b
            pltpu.make_async_copy(h_hbm.at[sidx.at[j]],
                                  rows[b], gsem[b]).wait()
            pltpu.sync_copy(rows[b], acc.at[didx.at[j]], add=True)

    for st in range(_CPT // _STAGE):
        _stage(st)
        if st == 0:
            pltpu.sync_copy(zeros_hbm, acc.at[pl.ds(sid * _RPT, _RPT)])
            plsc.subcore_barrier()
        _drain(st)

    plsc.subcore_barrier()
    pltpu.sync_copy(acc.at[pl.ds(sid * _RPT, _RPT)],
                    out_hbm.at[cid, pl.ds(sid * _RPT, _RPT)])


_BR = 1000
_GRID = _N // _BR


def _mlp_body(h_ref, p0_ref, p1_ref, w1_ref, b1_ref, w2_ref, b2_ref,
              g_ref, be_ref, out_ref, *, residual, final):
    h = h_ref[...]
    z = h + p0_ref[0] + p1_ref[0]
    a = lax.dot_general(z, w1_ref[...], (((1,), (1,)), ((), ())),
                        preferred_element_type=jnp.float32) + b1_ref[...]
    a = jnp.maximum(a, 0.0)
    y = lax.dot_general(a, w2_ref[...], (((1,), (1,)), ((), ())),
                        preferred_element_type=jnp.float32) + b2_ref[...]
    if residual:
        y = y + h
    m = jnp.mean(y, axis=1, keepdims=True)
    v = jnp.mean((y - m) ** 2, axis=1, keepdims=True)
    yn = (y - m) * lax.rsqrt(v + 1e-5) * g_ref[...] + be_ref[...]
    if final:
        @pl.when(pl.program_id(0) == 0)
        def _():
            out_ref[...] = jnp.zeros_like(out_ref)
        out_ref[...] += jnp.sum(yn, axis=0, keepdims=True)
    else:
        out_ref[...] = yn


def _mlp_tc(h, partials, w1, b1, w2, b2, g, be, *, residual, final):
    if final:
        out_shape = jax.ShapeDtypeStruct((1, _D), jnp.float32)
        out_spec = pl.BlockSpec((1, _D), lambda i: (0, 0))
    else:
        out_shape = jax.ShapeDtypeStruct((_N, _D), jnp.float32)
        out_spec = pl.BlockSpec((_BR, _D), lambda i: (i, 0))
    full = lambda s: pl.BlockSpec(s, lambda i: (0,) * len(s))
    return pl.pallas_call(
        functools.partial(_mlp_body, residual=residual, final=final),
        grid=(_GRID,),
        in_specs=[
            pl.BlockSpec((_BR, _D), lambda i: (i, 0)),
            pl.BlockSpec((1, _BR, _D), lambda i: (0, i, 0)),
            pl.BlockSpec((1, _BR, _D), lambda i: (1, i, 0)),
            full((_D, _D)), full((1, _D)),
            full((_D, _D)), full((1, _D)),
            full((1, _D)), full((1, _D)),
        ],
        out_specs=out_spec,
        out_shape=out_shape,
    )(h, partials, partials, w1, b1.reshape(1, _D), w2, b2.reshape(1, _D),
      g.reshape(1, _D), be.reshape(1, _D))


def kernel(x, edge_index, W1_0, b1_0, W2_0, b2_0, g_0, be_0,
           W1_1, b1_1, W2_1, b2_1, g_1, be_1,
           W1_2, b1_2, W2_2, b2_2, g_2, be_2):
    src = edge_index[0]
    dst = edge_index[1]
    pad = _EPAD - _E
    ar = jnp.arange(pad, dtype=jnp.int32)
    src_p = jnp.concatenate([src, (ar * 37) % _N])
    dst_p = jnp.concatenate([dst, _N + ar % (_NPAD - _N)])
    src3 = src_p.reshape(_NW, _CPT, _CHUNK)
    dst3 = dst_p.reshape(_NW, _CPT, _CHUNK)
    zeros = jnp.zeros((_RPT, _D), jnp.float32)

    params = [(W1_0, b1_0, W2_0, b2_0, g_0, be_0),
              (W1_1, b1_1, W2_1, b2_1, g_1, be_1),
              (W1_2, b1_2, W2_2, b2_2, g_2, be_2)]
    h = x
    for i, (w1, b1, w2, b2, g, be) in enumerate(params):
        partials = _segsum_sc(h, src3, dst3, zeros)
        h = _mlp_tc(h, partials, w1, b1, w2, b2, g, be,
                    residual=(i < 2), final=(i == 2))
    return h


# speedup vs baseline: 1.0253x; 1.0253x over previous
import functools

import jax
import jax.numpy as jnp
from jax import lax
from jax.experimental import pallas as pl
from jax.experimental.pallas import tpu as pltpu
from jax.experimental.pallas import tpu_sc as plsc

_N = 10000
_E = 320000
_D = 128

_NC = 2
_NS = 16
_NW = _NC * _NS
_CHUNK = 80
_CPT = 128
_EPAD = _NW * _CPT * _CHUNK
_NPAD = 10240
_RPT = _NPAD // _NS
_NBUF = 4
_STAGE = 32

_mesh = plsc.VectorSubcoreMesh(core_axis_name="c", subcore_axis_name="s")


@functools.partial(
    pl.kernel,
    out_type=jax.ShapeDtypeStruct((_NC, _NPAD, _D), jnp.float32),
    mesh=_mesh,
    scratch_types=[
        pltpu.VMEM((_STAGE, _CHUNK), jnp.int32),
        pltpu.VMEM((_STAGE, _CHUNK), jnp.int32),
        [pltpu.VMEM((_CHUNK, _D), jnp.float32)] * _NBUF,
        pltpu.VMEM_SHARED((_NPAD, _D), jnp.float32),
        [pltpu.SemaphoreType.DMA] * _NBUF,
    ],
)
def _segsum_sc(h_hbm, src_hbm, dst_hbm, zeros_hbm, out_hbm,
               sidx, didx, rows, acc, gsem):
    cid = lax.axis_index("c")
    sid = lax.axis_index("s")
    w = cid * _NS + sid

    def _stage(st):
        pltpu.sync_copy(src_hbm.at[w, pl.ds(st * _STAGE, _STAGE)], sidx)
        pltpu.sync_copy(dst_hbm.at[w, pl.ds(st * _STAGE, _STAGE)], didx)
        for b in range(_NBUF):
            pltpu.async_copy(h_hbm.at[sidx.at[b]], rows[b], gsem[b])

    def _drain(st):
        @pl.loop(0, _STAGE - _NBUF, step=_NBUF)
        def _chunks(j):
            for b in range(_NBUF):
                pltpu.make_async_copy(h_hbm.at[sidx.at[j + b]],
                                      rows[b], gsem[b]).wait()
                pltpu.sync_copy(rows[b], acc.at[didx.at[j + b]], add=True)
                pltpu.async_copy(h_hbm.at[sidx.at[j + _NBUF + b]],
                                 rows[b], gsem[b])
        for b in range(_NBUF):
            j = _STAGE - _NBUF + b
            pltpu.make_async_copy(h_hbm.at[sidx.at[j]],
                                  rows[b], gsem[b]).wait()
            pltpu.sync_copy(rows[b], acc.at[didx.at[j]], add=True)

    for st in range(_CPT // _STAGE):
        _stage(st)
        if st == 0:
            pltpu.sync_copy(zeros_hbm, acc.at[pl.ds(sid * _RPT, _RPT)])
            plsc.subcore_barrier()
        _drain(st)

    plsc.subcore_barrier()
    pltpu.sync_copy(acc.at[pl.ds(sid * _RPT, _RPT)],
                    out_hbm.at[cid, pl.ds(sid * _RPT, _RPT)])


_BR = 2000
_GRID = _N // _BR


def _mlp_body(h_ref, p0_ref, p1_ref, w1_ref, b1_ref, w2_ref, b2_ref,
              g_ref, be_ref, out_ref, *, residual, final):
    h = h_ref[...]
    z = h + p0_ref[0] + p1_ref[0]
    a = lax.dot_general(z, w1_ref[...], (((1,), (1,)), ((), ())),
                        preferred_element_type=jnp.float32) + b1_ref[...]
    a = jnp.maximum(a, 0.0)
    y = lax.dot_general(a, w2_ref[...], (((1,), (1,)), ((), ())),
                        preferred_element_type=jnp.float32) + b2_ref[...]
    if residual:
        y = y + h
    m = jnp.mean(y, axis=1, keepdims=True)
    v = jnp.mean((y - m) ** 2, axis=1, keepdims=True)
    yn = (y - m) * lax.rsqrt(v + 1e-5) * g_ref[...] + be_ref[...]
    if final:
        @pl.when(pl.program_id(0) == 0)
        def _():
            out_ref[...] = jnp.zeros_like(out_ref)
        out_ref[...] += jnp.sum(yn, axis=0, keepdims=True)
    else:
        out_ref[...] = yn


def _mlp_tc(h, partials, w1, b1, w2, b2, g, be, *, residual, final):
    if final:
        out_shape = jax.ShapeDtypeStruct((1, _D), jnp.float32)
        out_spec = pl.BlockSpec((1, _D), lambda i: (0, 0))
    else:
        out_shape = jax.ShapeDtypeStruct((_N, _D), jnp.float32)
        out_spec = pl.BlockSpec((_BR, _D), lambda i: (i, 0))
    full = lambda s: pl.BlockSpec(s, lambda i: (0,) * len(s))
    return pl.pallas_call(
        functools.partial(_mlp_body, residual=residual, final=final),
        grid=(_GRID,),
        in_specs=[
            pl.BlockSpec((_BR, _D), lambda i: (i, 0)),
            pl.BlockSpec((1, _BR, _D), lambda i: (0, i, 0)),
            pl.BlockSpec((1, _BR, _D), lambda i: (1, i, 0)),
            full((_D, _D)), full((1, _D)),
            full((_D, _D)), full((1, _D)),
            full((1, _D)), full((1, _D)),
        ],
        out_specs=out_spec,
        out_shape=out_shape,
    )(h, partials, partials, w1, b1.reshape(1, _D), w2, b2.reshape(1, _D),
      g.reshape(1, _D), be.reshape(1, _D))


def kernel(x, edge_index, W1_0, b1_0, W2_0, b2_0, g_0, be_0,
           W1_1, b1_1, W2_1, b2_1, g_1, be_1,
           W1_2, b1_2, W2_2, b2_2, g_2, be_2):
    src = edge_index[0]
    dst = edge_index[1]
    pad = _EPAD - _E
    ar = jnp.arange(pad, dtype=jnp.int32)
    src_p = jnp.concatenate([src, (ar * 37) % _N])
    dst_p = jnp.concatenate([dst, _N + ar % (_NPAD - _N)])
    src3 = src_p.reshape(_NW, _CPT, _CHUNK)
    dst3 = dst_p.reshape(_NW, _CPT, _CHUNK)
    zeros = jnp.zeros((_RPT, _D), jnp.float32)

    params = [(W1_0, b1_0, W2_0, b2_0, g_0, be_0),
              (W1_1, b1_1, W2_1, b2_1, g_1, be_1),
              (W1_2, b1_2, W2_2, b2_2, g_2, be_2)]
    h = x
    for i, (w1, b1, w2, b2, g, be) in enumerate(params):
        partials = _segsum_sc(h, src3, dst3, zeros)
        h = _mlp_tc(h, partials, w1, b1, w2, b2, g, be,
                    residual=(i < 2), final=(i == 2))
    return h


# speedup vs baseline: 1.0384x; 1.0127x over previous
import functools

import jax
import jax.numpy as jnp
from jax import lax
from jax.experimental import pallas as pl
from jax.experimental.pallas import tpu as pltpu
from jax.experimental.pallas import tpu_sc as plsc

_N = 10000
_E = 320000
_D = 128

_NC = 2
_NS = 16
_NW = _NC * _NS
_CHUNK = 80
_CPT = 128
_EPAD = _NW * _CPT * _CHUNK
_NPAD = 10240
_RPT = _NPAD // _NS
_NBUF = 4
_STAGE = 32

_mesh = plsc.VectorSubcoreMesh(core_axis_name="c", subcore_axis_name="s")


@functools.partial(
    pl.kernel,
    out_type=jax.ShapeDtypeStruct((_NC, _NPAD, _D), jnp.float32),
    mesh=_mesh,
    scratch_types=[
        pltpu.VMEM((_STAGE, _CHUNK), jnp.int32),
        pltpu.VMEM((_STAGE, _CHUNK), jnp.int32),
        [pltpu.VMEM((_CHUNK, _D), jnp.float32)] * _NBUF,
        pltpu.VMEM_SHARED((_NPAD, _D), jnp.float32),
        [pltpu.SemaphoreType.DMA] * _NBUF,
    ],
)
def _segsum_sc(h_hbm, src_hbm, dst_hbm, zeros_hbm, out_hbm,
               sidx, didx, rows, acc, gsem):
    cid = lax.axis_index("c")
    sid = lax.axis_index("s")
    w = cid * _NS + sid

    def _stage(st):
        pltpu.sync_copy(src_hbm.at[w, pl.ds(st * _STAGE, _STAGE)], sidx)
        pltpu.sync_copy(dst_hbm.at[w, pl.ds(st * _STAGE, _STAGE)], didx)
        for b in range(_NBUF):
            pltpu.async_copy(h_hbm.at[sidx.at[b]], rows[b], gsem[b])

    def _drain(st):
        @pl.loop(0, _STAGE - _NBUF, step=_NBUF)
        def _chunks(j):
            for b in range(_NBUF):
                pltpu.make_async_copy(h_hbm.at[sidx.at[j + b]],
                                      rows[b], gsem[b]).wait()
                pltpu.sync_copy(rows[b], acc.at[didx.at[j + b]], add=True)
                pltpu.async_copy(h_hbm.at[sidx.at[j + _NBUF + b]],
                                 rows[b], gsem[b])
        for b in range(_NBUF):
            j = _STAGE - _NBUF + b
            pltpu.make_async_copy(h_hbm.at[sidx.at[j]],
                                  rows[b], gsem[b]).wait()
            pltpu.sync_copy(rows[b], acc.at[didx.at[j]], add=True)

    for st in range(_CPT // _STAGE):
        _stage(st)
        if st == 0:
            pltpu.sync_copy(zeros_hbm, acc.at[pl.ds(sid * _RPT, _RPT)])
            plsc.subcore_barrier()
        _drain(st)

    plsc.subcore_barrier()
    pltpu.sync_copy(acc.at[pl.ds(sid * _RPT, _RPT)],
                    out_hbm.at[cid, pl.ds(sid * _RPT, _RPT)])


_BR = 5000
_GRID = _N // _BR


def _mlp_body(h_ref, p0_ref, p1_ref, w1_ref, b1_ref, w2_ref, b2_ref,
              g_ref, be_ref, out_ref, *, residual, final):
    h = h_ref[...]
    z = h + p0_ref[0] + p1_ref[0]
    a = lax.dot_general(z, w1_ref[...], (((1,), (1,)), ((), ())),
                        preferred_element_type=jnp.float32) + b1_ref[...]
    a = jnp.maximum(a, 0.0)
    y = lax.dot_general(a, w2_ref[...], (((1,), (1,)), ((), ())),
                        preferred_element_type=jnp.float32) + b2_ref[...]
    if residual:
        y = y + h
    m = jnp.mean(y, axis=1, keepdims=True)
    v = jnp.mean((y - m) ** 2, axis=1, keepdims=True)
    yn = (y - m) * lax.rsqrt(v + 1e-5) * g_ref[...] + be_ref[...]
    if final:
        @pl.when(pl.program_id(0) == 0)
        def _():
            out_ref[...] = jnp.zeros_like(out_ref)
        out_ref[...] += jnp.sum(yn, axis=0, keepdims=True)
    else:
        out_ref[...] = yn


def _mlp_tc(h, partials, w1, b1, w2, b2, g, be, *, residual, final):
    if final:
        out_shape = jax.ShapeDtypeStruct((1, _D), jnp.float32)
        out_spec = pl.BlockSpec((1, _D), lambda i: (0, 0))
    else:
        out_shape = jax.ShapeDtypeStruct((_N, _D), jnp.float32)
        out_spec = pl.BlockSpec((_BR, _D), lambda i: (i, 0))
    full = lambda s: pl.BlockSpec(s, lambda i: (0,) * len(s))
    return pl.pallas_call(
        functools.partial(_mlp_body, residual=residual, final=final),
        grid=(_GRID,),
        in_specs=[
            pl.BlockSpec((_BR, _D), lambda i: (i, 0)),
            pl.BlockSpec((1, _BR, _D), lambda i: (0, i, 0)),
            pl.BlockSpec((1, _BR, _D), lambda i: (1, i, 0)),
            full((_D, _D)), full((1, _D)),
            full((_D, _D)), full((1, _D)),
            full((1, _D)), full((1, _D)),
        ],
        out_specs=out_spec,
        out_shape=out_shape,
    )(h, partials, partials, w1, b1.reshape(1, _D), w2, b2.reshape(1, _D),
      g.reshape(1, _D), be.reshape(1, _D))


def kernel(x, edge_index, W1_0, b1_0, W2_0, b2_0, g_0, be_0,
           W1_1, b1_1, W2_1, b2_1, g_1, be_1,
           W1_2, b1_2, W2_2, b2_2, g_2, be_2):
    src = edge_index[0]
    dst = edge_index[1]
    pad = _EPAD - _E
    ar = jnp.arange(pad, dtype=jnp.int32)
    src_p = jnp.concatenate([src, (ar * 37) % _N])
    dst_p = jnp.concatenate([dst, _N + ar % (_NPAD - _N)])
    src3 = src_p.reshape(_NW, _CPT, _CHUNK)
    dst3 = dst_p.reshape(_NW, _CPT, _CHUNK)
    zeros = jnp.zeros((_RPT, _D), jnp.float32)

    params = [(W1_0, b1_0, W2_0, b2_0, g_0, be_0),
              (W1_1, b1_1, W2_1, b2_1, g_1, be_1),
              (W1_2, b1_2, W2_2, b2_2, g_2, be_2)]
    h = x
    for i, (w1, b1, w2, b2, g, be) in enumerate(params):
        partials = _segsum_sc(h, src3, dst3, zeros)
        h = _mlp_tc(h, partials, w1, b1, w2, b2, g, be,
                    residual=(i < 2), final=(i == 2))
    return h
